# baseline (device time: 89971 ns/iter reference)
import os

import jax
import jax.numpy as jnp
from jax import lax
from jax.experimental import pallas as pl
from jax.experimental.pallas import tpu as pltpu

_SKIP_COMPUTE = os.environ.get("KSKIP_COMPUTE") == "1"
_SKIP_AR = os.environ.get("KSKIP_AR") == "1"

N_DEV = 4
B, SQ, DMODEL = 2, 512, 768
HQ_TOT, DH = 32, 64
H_LOC = HQ_TOT // N_DEV
HF = H_LOC * DH
SKV = 512
NQ = 4
QT = SKV // NQ
BLK = 64
NEG = -1e9
CHK = SQ // N_DEV


def kernel(x, Wq, K_ext, V_ext, Wo):
    K2 = K_ext.reshape(B, SKV, HQ_TOT * DH).astype(jnp.bfloat16)
    V2 = V_ext.reshape(B, SKV, HQ_TOT * DH).astype(jnp.bfloat16)
    xb = x.astype(jnp.bfloat16)
    Wqb = Wq.astype(jnp.bfloat16)
    Wob = Wo.astype(jnp.bfloat16)

    def body(x_ref, wq_ref, k_ref, v_ref, wo_ref, out_ref,
             kvfull, kv_send, kv_buf, relay_buf,
             rs_stage, rs_recv, ag_stage, ag_recv,
             full_sems, scat_send_sems, scat_recv_sems, relay_recv_sems,
             fw_send_sems,
             rs_send_sems, rs_recv_sems, ag_send_sems, ag_recv_sems):
        my = lax.axis_index("i")
        is_src = my == 0
        own = (my + 1) % N_DEV

        bar = pltpu.get_barrier_semaphore()
        for off in range(1, N_DEV):
            peer = (my + off) % N_DEV
            pl.semaphore_signal(
                bar, inc=1,
                device_id=(peer,), device_id_type=pl.DeviceIdType.MESH,
            )

        @pl.when(is_src)
        def _():
            for qt in range(NQ):
                rows = pl.ds(qt * QT, QT)
                pltpu.make_async_copy(
                    k_ref.at[:, rows, :], kvfull.at[0, :, rows, :],
                    full_sems.at[qt, 0]).start()
                pltpu.make_async_copy(
                    v_ref.at[:, rows, :], kvfull.at[1, :, rows, :],
                    full_sems.at[qt, 1]).start()

        wo_bf = wo_ref[:]
        qs = []
        for b in range(B):
            q = jnp.dot(x_ref[b], wq_ref[:],
                        preferred_element_type=jnp.float32)
            qs.append((q * 0.125).reshape(SQ, H_LOC, DH).astype(jnp.bfloat16))

        pl.semaphore_wait(bar, N_DEV - 1)

        @pl.when(is_src)
        def _():
            for qt in range(NQ):
                rows = pl.ds(qt * QT, QT)
                pltpu.make_async_copy(
                    k_ref.at[:, rows, :], kvfull.at[0, :, rows, :],
                    full_sems.at[qt, 0]).wait()
                pltpu.make_async_copy(
                    v_ref.at[:, rows, :], kvfull.at[1, :, rows, :],
                    full_sems.at[qt, 1]).wait()
                kvh = kvfull[:, :, qt * QT:(qt + 1) * QT, :]
                for d in range(1, N_DEV):
                    kv_send[qt, d - 1] = kvh[:, :, :, d * HF:(d + 1) * HF]
                sends = [
                    (kv_send.at[qt, 0], kv_buf.at[:, :, rows, :],
                     scat_recv_sems, 1),
                    (kv_send.at[qt, 1, 0], relay_buf.at[qt],
                     relay_recv_sems, 1),
                    (kv_send.at[qt, 2], kv_buf.at[:, :, rows, :],
                     scat_recv_sems, 3),
                    (kv_send.at[qt, 1, 1], relay_buf.at[qt],
                     relay_recv_sems, 3),
                ]
                for i, (src, dst, rsem, dev) in enumerate(sends):
                    pltpu.make_async_remote_copy(
                        src_ref=src, dst_ref=dst,
                        send_sem=scat_send_sems.at[qt, i],
                        recv_sem=rsem.at[qt],
                        device_id=(dev,),
                        device_id_type=pl.DeviceIdType.MESH,
                    ).start()
                kv_buf[:, :, rows, :] = kvh[:, :, :, 0:HF]

        def relay_wait_fwd(qt, piece):
            rows = pl.ds(qt * QT, QT)
            pltpu.make_async_remote_copy(
                src_ref=relay_buf.at[qt], dst_ref=relay_buf.at[qt],
                send_sem=fw_send_sems.at[qt],
                recv_sem=relay_recv_sems.at[qt],
                device_id=(0,), device_id_type=pl.DeviceIdType.MESH,
            ).wait_recv()
            pltpu.make_async_remote_copy(
                src_ref=relay_buf.at[qt],
                dst_ref=kv_buf.at[piece, :, rows, :],
                send_sem=fw_send_sems.at[qt],
                recv_sem=(scat_recv_sems if piece == 0
                          else relay_recv_sems).at[qt],
                device_id=(2,), device_id_type=pl.DeviceIdType.MESH,
            ).start()

        def main_wait(qt):
            rows = pl.ds(qt * QT, QT)
            pltpu.make_async_remote_copy(
                src_ref=kv_buf.at[:, :, rows, :],
                dst_ref=kv_buf.at[:, :, rows, :],
                send_sem=fw_send_sems.at[qt],
                recv_sem=scat_recv_sems.at[qt],
                device_id=(0,), device_id_type=pl.DeviceIdType.MESH,
            ).wait_recv()

        def mid_wait(qt):
            rows = pl.ds(qt * QT, QT)
            for piece, rsem in ((0, scat_recv_sems), (1, relay_recv_sems)):
                pltpu.make_async_remote_copy(
                    src_ref=kv_buf.at[piece, :, rows, :],
                    dst_ref=kv_buf.at[piece, :, rows, :],
                    send_sem=fw_send_sems.at[qt],
                    recv_sem=rsem.at[qt],
                    device_id=(0,), device_id_type=pl.DeviceIdType.MESH,
                ).wait_recv()

        qb_base = lax.broadcasted_iota(jnp.int32, (CHK, SKV), 0)
        kb_full = lax.broadcasted_iota(jnp.int32, (CHK, SKV), 1) // BLK

        for c in range(N_DEV):
            @pl.when(my == 1)
            def _():
                relay_wait_fwd(c, 0)
                main_wait(c)
            @pl.when(my == 3)
            def _():
                relay_wait_fwd(c, 1)
                main_wait(c)
            @pl.when(my == 2)
            def _():
                mid_wait(c)
            kl = CHK * (c + 1)
            mask = kb_full[:, :kl] <= (qb_base[:, :kl] + c * CHK) // BLK
            if _SKIP_COMPUTE:
                for b in range(B):
                    z = jnp.zeros((CHK, DMODEL), jnp.float32)
                    out_ref[b, c * CHK:(c + 1) * CHK, :] = z
                    rs_stage[c, b] = z.astype(jnp.bfloat16)
            for b in ([] if _SKIP_COMPUTE else range(B)):
                ctx_parts = []
                for h in range(H_LOC):
                    kh = kv_buf[0, b, 0:kl, h * DH:(h + 1) * DH]
                    vh = kv_buf[1, b, 0:kl, h * DH:(h + 1) * DH]
                    qch = qs[b][c * CHK:(c + 1) * CHK, h, :]
                    s = lax.dot_general(
                        qch, kh, (((1,), (1,)), ((), ())),
                        preferred_element_type=jnp.float32)
                    w = jnp.where(mask, jnp.exp(s), 0.0)
                    inv = 1.0 / jnp.sum(w, axis=-1, keepdims=True)
                    ctx_parts.append(
                        jnp.dot(w.astype(jnp.bfloat16), vh,
                                preferred_element_type=jnp.float32) * inv)
                ctx = jnp.concatenate(ctx_parts, axis=-1)
                part = jnp.dot(ctx.astype(jnp.bfloat16), wo_bf,
                               preferred_element_type=jnp.float32)
                out_ref[b, c * CHK:(c + 1) * CHK, :] = part
                rs_stage[c, b] = part.astype(jnp.bfloat16)
            if _SKIP_AR:
                continue
            p = (c - 1) % N_DEV
            @pl.when(jnp.int32(c) != own)
            def _():
                slot = (my - p) % N_DEV - 1
                pltpu.make_async_remote_copy(
                    src_ref=rs_stage.at[c],
                    dst_ref=rs_recv.at[slot],
                    send_sem=rs_send_sems.at[c],
                    recv_sem=rs_recv_sems.at[slot],
                    device_id=(p,), device_id_type=pl.DeviceIdType.MESH,
                ).start()

        for slot in ([] if _SKIP_AR else range(N_DEV - 1)):
            pltpu.make_async_remote_copy(
                src_ref=rs_recv.at[slot], dst_ref=rs_recv.at[slot],
                send_sem=rs_send_sems.at[0],
                recv_sem=rs_recv_sems.at[slot],
                device_id=(0,), device_id_type=pl.DeviceIdType.MESH,
            ).wait_recv()
        if not _SKIP_AR:
            red = out_ref[:, pl.ds(own * CHK, CHK), :]
            for slot in range(N_DEV - 1):
                red = red + rs_recv[slot].astype(jnp.float32)
            out_ref[:, pl.ds(own * CHK, CHK), :] = red
            ag_stage[:] = red.astype(jnp.bfloat16)

        for off in ([] if _SKIP_AR else range(1, N_DEV)):
            p = (my + off) % N_DEV
            pltpu.make_async_remote_copy(
                src_ref=ag_stage,
                dst_ref=ag_recv.at[N_DEV - 1 - off],
                send_sem=ag_send_sems.at[off - 1],
                recv_sem=ag_recv_sems.at[N_DEV - 1 - off],
                device_id=(p,), device_id_type=pl.DeviceIdType.MESH,
            ).start()
        for off in ([] if _SKIP_AR else range(1, N_DEV)):
            slot = off - 1
            pltpu.make_async_remote_copy(
                src_ref=ag_recv.at[slot], dst_ref=ag_recv.at[slot],
                send_sem=ag_send_sems.at[0],
                recv_sem=ag_recv_sems.at[slot],
                device_id=(0,), device_id_type=pl.DeviceIdType.MESH,
            ).wait_recv()
            src_chunk = (my + off + 1) % N_DEV
            out_ref[:, pl.ds(src_chunk * CHK, CHK), :] = (
                ag_recv[slot].astype(jnp.float32))

        @pl.when(is_src)
        def _():
            for qt in range(NQ):
                rows = pl.ds(qt * QT, QT)
                drains = [
                    (kv_send.at[qt, 0], kv_buf.at[:, :, rows, :], 1),
                    (kv_send.at[qt, 1, 0], relay_buf.at[qt], 1),
                    (kv_send.at[qt, 2], kv_buf.at[:, :, rows, :], 3),
                    (kv_send.at[qt, 1, 1], relay_buf.at[qt], 3),
                ]
                for i, (src, dst, dev) in enumerate(drains):
                    pltpu.make_async_remote_copy(
                        src_ref=src, dst_ref=dst,
                        send_sem=scat_send_sems.at[qt, i],
                        recv_sem=scat_recv_sems.at[qt],
                        device_id=(dev,),
                        device_id_type=pl.DeviceIdType.MESH,
                    ).wait_send()

        @pl.when((my == 1) | (my == 3))
        def _():
            for qt in range(NQ):
                pltpu.make_async_remote_copy(
                    src_ref=relay_buf.at[qt],
                    dst_ref=kv_buf.at[0, :, pl.ds(qt * QT, QT), :],
                    send_sem=fw_send_sems.at[qt],
                    recv_sem=relay_recv_sems.at[qt],
                    device_id=(2,), device_id_type=pl.DeviceIdType.MESH,
                ).wait_send()
        for c in ([] if _SKIP_AR else range(N_DEV)):
            @pl.when(jnp.int32(c) != own)
            def _():
                pltpu.make_async_remote_copy(
                    src_ref=rs_stage.at[c], dst_ref=rs_recv.at[0],
                    send_sem=rs_send_sems.at[c],
                    recv_sem=rs_recv_sems.at[0],
                    device_id=(0,), device_id_type=pl.DeviceIdType.MESH,
                ).wait_send()
        for off in ([] if _SKIP_AR else range(1, N_DEV)):
            pltpu.make_async_remote_copy(
                src_ref=ag_stage, dst_ref=ag_recv.at[0],
                send_sem=ag_send_sems.at[off - 1],
                recv_sem=ag_recv_sems.at[0],
                device_id=(0,), device_id_type=pl.DeviceIdType.MESH,
            ).wait_send()

    out_shape = jax.ShapeDtypeStruct((B, SQ, DMODEL), jnp.float32)
    return pl.pallas_call(
        body,
        out_shape=out_shape,
        in_specs=[
            pl.BlockSpec(memory_space=pltpu.VMEM),
            pl.BlockSpec(memory_space=pltpu.VMEM),
            pl.BlockSpec(memory_space=pl.ANY),
            pl.BlockSpec(memory_space=pl.ANY),
            pl.BlockSpec(memory_space=pltpu.VMEM),
        ],
        out_specs=pl.BlockSpec(memory_space=pltpu.VMEM),
        scratch_shapes=[
            pltpu.VMEM((2, B, SKV, HQ_TOT * DH), jnp.bfloat16),
            pltpu.VMEM((NQ, N_DEV - 1, 2, B, QT, HF), jnp.bfloat16),
            pltpu.VMEM((2, B, SKV, HF), jnp.bfloat16),
            pltpu.VMEM((NQ, B, QT, HF), jnp.bfloat16),
            pltpu.VMEM((N_DEV, B, CHK, DMODEL), jnp.bfloat16),
            pltpu.VMEM((N_DEV - 1, B, CHK, DMODEL), jnp.bfloat16),
            pltpu.VMEM((B, CHK, DMODEL), jnp.bfloat16),
            pltpu.VMEM((N_DEV - 1, B, CHK, DMODEL), jnp.bfloat16),
            pltpu.SemaphoreType.DMA((NQ, 2)),
            pltpu.SemaphoreType.DMA((NQ, 4)),
            pltpu.SemaphoreType.DMA((NQ,)),
            pltpu.SemaphoreType.DMA((NQ,)),
            pltpu.SemaphoreType.DMA((NQ,)),
            pltpu.SemaphoreType.DMA((N_DEV,)),
            pltpu.SemaphoreType.DMA((N_DEV - 1,)),
            pltpu.SemaphoreType.DMA((N_DEV - 1,)),
            pltpu.SemaphoreType.DMA((N_DEV - 1,)),
        ],
        compiler_params=pltpu.CompilerParams(
            collective_id=0,
            vmem_limit_bytes=120 * 1024 * 1024,
        ),
    )(xb, Wqb, K2, V2, Wob)
